# Initial kernel scaffold; baseline (speedup 1.0000x reference)
import jax
import jax.numpy as jnp
from jax.experimental import pallas as pl


def kernel(i_emb, o_emb, target_wids, context_wids, neg_wids):
    raise NotImplementedError("write your pallas kernel here")


# speedup vs baseline: 1.9938x; 1.9938x over previous
import functools

import jax
import jax.numpy as jnp
from jax import lax
from jax.experimental import pallas as pl
from jax.experimental.pallas import tpu as pltpu
from jax.experimental.pallas import tpu_sc as plsc

V = 100000
D = 64
K = 10
CTX = 10
B = 16384
NC = 2
NS = 16
NW = NC * NS
BPW = B // NW
C = 64
N_CHUNKS = BPW // C
S = K + 1
SP = 16


def _sc_scores_kernel(i_emb, o_emb, tgt_hbm, ctx_hbm, neg_hbm, out_hbm,
                      tgt_idx, ctx_idx, neg_idx,
                      tgt_rows, ctx_rows, neg_rows, scores, pbuf,
                      sem_t, sem_c, sem_n):
    wid = lax.axis_index("s") * NC + lax.axis_index("c")
    base = wid * BPW

    def chunk_body(t, carry):
        row0 = base + t * C
        pltpu.sync_copy(tgt_hbm.at[pl.ds(row0, C)], tgt_idx)
        pltpu.sync_copy(ctx_hbm.at[pl.ds(row0 * CTX, C * CTX)], ctx_idx)
        pltpu.sync_copy(neg_hbm.at[pl.ds(row0 * K, C * K)], neg_idx)
        g_t = pltpu.async_copy(o_emb.at[tgt_idx], tgt_rows, sem_t)
        g_c = pltpu.async_copy(i_emb.at[ctx_idx], ctx_rows, sem_c)
        g_n = pltpu.async_copy(o_emb.at[neg_idx], neg_rows, sem_n)
        g_t.wait()
        g_c.wait()
        g_n.wait()

        lane = lax.iota(jnp.int32, 16)
        lane_sel = jnp.where(lane < S, lane, 0)
        sgn = jnp.where(lane == 0, 1.0, -1.0)
        valid = lane < S
        col_idx = [jnp.full((16,), j, jnp.int32) for j in range(16)]

        def row_body(c, carry2):
            rc = c * CTX
            avg = []
            for q in range(D // 16):
                a = ctx_rows[rc, pl.ds(q * 16, 16)]
                for j in range(1, CTX):
                    a = a + ctx_rows[rc + j, pl.ds(q * 16, 16)]
                avg.append(a * (1.0 / CTX))
            p = tgt_rows[c, pl.ds(0, 16)] * avg[0]
            for q in range(1, D // 16):
                p = p + tgt_rows[c, pl.ds(q * 16, 16)] * avg[q]
            pbuf[0, :] = p
            rn = c * K
            for k in range(K):
                p = neg_rows[rn + k, pl.ds(0, 16)] * avg[0]
                for q in range(1, D // 16):
                    p = p + neg_rows[rn + k, pl.ds(q * 16, 16)] * avg[q]
                pbuf[k + 1, :] = p
            acc = plsc.load_gather(pbuf, [lane_sel, col_idx[0]])
            for j in range(1, 16):
                acc = acc + plsc.load_gather(pbuf, [lane_sel, col_idx[j]])
            scores[c, :] = jnp.where(valid, sgn * acc, 0.0)
            return carry2

        lax.fori_loop(0, C, row_body, 0)
        pltpu.sync_copy(scores, out_hbm.at[pl.ds(row0, C), :])
        return carry

    lax.fori_loop(0, N_CHUNKS, chunk_body, 0)


_sc_scores = functools.partial(
    pl.kernel,
    mesh=plsc.VectorSubcoreMesh(core_axis_name="c", subcore_axis_name="s"),
    compiler_params=pltpu.CompilerParams(
        needs_layout_passes=False, use_tc_tiling_on_sc=False
    ),
    out_type=jax.ShapeDtypeStruct((B, SP), jnp.float32),
    scratch_types=[
        pltpu.VMEM((C,), jnp.int32),
        pltpu.VMEM((C * CTX,), jnp.int32),
        pltpu.VMEM((C * K,), jnp.int32),
        pltpu.VMEM((C, D), jnp.float32),
        pltpu.VMEM((C * CTX, D), jnp.float32),
        pltpu.VMEM((C * K, D), jnp.float32),
        pltpu.VMEM((C, SP), jnp.float32),
        pltpu.VMEM((16, 16), jnp.float32),
        pltpu.SemaphoreType.DMA,
        pltpu.SemaphoreType.DMA,
        pltpu.SemaphoreType.DMA,
    ],
)(_sc_scores_kernel)


def _tc_loss_kernel(x_ref, o_ref):
    x = x_ref[...]
    valid = lax.broadcasted_iota(jnp.int32, x.shape, 1) % SP < S
    sp = jnp.maximum(-x, 0.0) + jnp.log1p(jnp.exp(-jnp.abs(x)))
    o_ref[0, 0] = jnp.sum(jnp.where(valid, sp, 0.0))


_tc_loss = pl.pallas_call(
    _tc_loss_kernel,
    out_shape=jax.ShapeDtypeStruct((1, 1), jnp.float32),
    out_specs=pl.BlockSpec(memory_space=pltpu.SMEM),
)


def kernel(i_emb, o_emb, target_wids, context_wids, neg_wids):
    tgt = target_wids.astype(jnp.int32)
    ctx = context_wids.astype(jnp.int32).reshape(-1)
    neg = neg_wids.astype(jnp.int32).reshape(-1)
    scores = _sc_scores(i_emb, o_emb, tgt, ctx, neg)
    loss = _tc_loss(scores.reshape(B * SP // 128, 128))
    return loss[0, 0]


# speedup vs baseline: 2.1440x; 1.0753x over previous
import functools

import jax
import jax.numpy as jnp
from jax import lax
from jax.experimental import pallas as pl
from jax.experimental.pallas import tpu as pltpu
from jax.experimental.pallas import tpu_sc as plsc

V = 100000
D = 64
K = 10
CTX = 10
B = 16384
NC = 2
NS = 16
NW = NC * NS
BPW = B // NW
C = 32
N_CHUNKS = BPW // C
S = K + 1
SP = 16


def _sc_scores_kernel(i_emb, o_emb, tgt_hbm, ctx_hbm, neg_hbm, out_hbm,
                      tgt_idx, ctx_idx, neg_idx,
                      tgt_rows, ctx_rows, neg_rows, scores, pbuf,
                      sem_t, sem_c, sem_n):
    wid = lax.axis_index("s") * NC + lax.axis_index("c")
    base = wid * BPW

    def fire(t, b):
        row0 = base + t * C
        pltpu.sync_copy(tgt_hbm.at[pl.ds(row0, C)], tgt_idx.at[b])
        pltpu.sync_copy(ctx_hbm.at[pl.ds(row0 * CTX, C * CTX)], ctx_idx.at[b])
        pltpu.sync_copy(neg_hbm.at[pl.ds(row0 * K, C * K)], neg_idx.at[b])
        pltpu.async_copy(o_emb.at[tgt_idx.at[b]], tgt_rows.at[b], sem_t.at[b])
        pltpu.async_copy(i_emb.at[ctx_idx.at[b]], ctx_rows.at[b], sem_c.at[b])
        pltpu.async_copy(o_emb.at[neg_idx.at[b]], neg_rows.at[b], sem_n.at[b])

    def drain(b):
        pltpu.make_async_copy(
            o_emb.at[tgt_idx.at[b]], tgt_rows.at[b], sem_t.at[b]).wait()
        pltpu.make_async_copy(
            i_emb.at[ctx_idx.at[b]], ctx_rows.at[b], sem_c.at[b]).wait()
        pltpu.make_async_copy(
            o_emb.at[neg_idx.at[b]], neg_rows.at[b], sem_n.at[b]).wait()

    lane = lax.iota(jnp.int32, 16)
    lane_sel = jnp.where(lane < S, lane, 0)
    sgn = jnp.where(lane == 0, 1.0, -1.0)
    valid = lane < S
    col_idx = [jnp.full((16,), j, jnp.int32) for j in range(16)]

    def compute(t, b):
        row0 = base + t * C

        def row_body(c, carry2):
            rc = c * CTX
            avg = []
            for q in range(D // 16):
                a = ctx_rows[b, rc, pl.ds(q * 16, 16)]
                for j in range(1, CTX):
                    a = a + ctx_rows[b, rc + j, pl.ds(q * 16, 16)]
                avg.append(a * (1.0 / CTX))
            p = tgt_rows[b, c, pl.ds(0, 16)] * avg[0]
            for q in range(1, D // 16):
                p = p + tgt_rows[b, c, pl.ds(q * 16, 16)] * avg[q]
            pbuf[0, :] = p
            rn = c * K
            for k in range(K):
                p = neg_rows[b, rn + k, pl.ds(0, 16)] * avg[0]
                for q in range(1, D // 16):
                    p = p + neg_rows[b, rn + k, pl.ds(q * 16, 16)] * avg[q]
                pbuf[k + 1, :] = p
            acc = plsc.load_gather(pbuf, [lane_sel, col_idx[0]])
            for j in range(1, 16):
                acc = acc + plsc.load_gather(pbuf, [lane_sel, col_idx[j]])
            scores[b, c, :] = jnp.where(valid, sgn * acc, 0.0)
            return carry2

        lax.fori_loop(0, C, row_body, 0)
        pltpu.sync_copy(scores.at[b], out_hbm.at[pl.ds(row0, C), :])

    fire(0, 0)

    def body(i, carry):
        t0 = 2 * i
        fire(t0 + 1, 1)
        drain(0)
        compute(t0, 0)

        @pl.when(i < N_CHUNKS // 2 - 1)
        def _():
            fire(t0 + 2, 0)

        drain(1)
        compute(t0 + 1, 1)
        return carry

    lax.fori_loop(0, N_CHUNKS // 2, body, 0)


_sc_scores = functools.partial(
    pl.kernel,
    mesh=plsc.VectorSubcoreMesh(core_axis_name="c", subcore_axis_name="s"),
    compiler_params=pltpu.CompilerParams(
        needs_layout_passes=False, use_tc_tiling_on_sc=False
    ),
    out_type=jax.ShapeDtypeStruct((B, SP), jnp.float32),
    scratch_types=[
        pltpu.VMEM((2, C), jnp.int32),
        pltpu.VMEM((2, C * CTX), jnp.int32),
        pltpu.VMEM((2, C * K), jnp.int32),
        pltpu.VMEM((2, C, D), jnp.float32),
        pltpu.VMEM((2, C * CTX, D), jnp.float32),
        pltpu.VMEM((2, C * K, D), jnp.float32),
        pltpu.VMEM((2, C, SP), jnp.float32),
        pltpu.VMEM((16, 16), jnp.float32),
        pltpu.SemaphoreType.DMA((2,)),
        pltpu.SemaphoreType.DMA((2,)),
        pltpu.SemaphoreType.DMA((2,)),
    ],
)(_sc_scores_kernel)


def _tc_loss_kernel(x_ref, o_ref):
    x = x_ref[...]
    valid = lax.broadcasted_iota(jnp.int32, x.shape, 1) % SP < S
    sp = jnp.maximum(-x, 0.0) + jnp.log1p(jnp.exp(-jnp.abs(x)))
    o_ref[0, 0] = jnp.sum(jnp.where(valid, sp, 0.0))


_tc_loss = pl.pallas_call(
    _tc_loss_kernel,
    out_shape=jax.ShapeDtypeStruct((1, 1), jnp.float32),
    out_specs=pl.BlockSpec(memory_space=pltpu.SMEM),
)


def kernel(i_emb, o_emb, target_wids, context_wids, neg_wids):
    tgt = target_wids.astype(jnp.int32)
    ctx = context_wids.astype(jnp.int32).reshape(-1)
    neg = neg_wids.astype(jnp.int32).reshape(-1)
    scores = _sc_scores(i_emb, o_emb, tgt, ctx, neg)
    loss = _tc_loss(scores.reshape(B * SP // 128, 128))
    return loss[0, 0]


# speedup vs baseline: 2.1699x; 1.0121x over previous
import functools

import jax
import jax.numpy as jnp
from jax import lax
from jax.experimental import pallas as pl
from jax.experimental.pallas import tpu as pltpu
from jax.experimental.pallas import tpu_sc as plsc

V = 100000
D = 64
K = 10
CTX = 10
B = 16384
NC = 2
NS = 16
NW = NC * NS
BPW = B // NW
C = 32
N_CHUNKS = BPW // C
S = K + 1


def _sc_loss_kernel(i_emb, o_emb, tgt_hbm, cn_hbm, out_hbm,
                    tgt_idx, ctx_idx, neg_idx,
                    tgt_rows, ctx_rows, neg_rows, pbuf, acc_buf,
                    sem_t, sem_c, sem_n):
    wid = lax.axis_index("s") * NC + lax.axis_index("c")
    base = wid * BPW

    def fire(t, b):
        row0 = base + t * C
        pltpu.sync_copy(tgt_hbm.at[pl.ds(row0, C)], tgt_idx.at[b])
        pltpu.sync_copy(cn_hbm.at[pl.ds(row0 * CTX, C * CTX)], ctx_idx.at[b])
        pltpu.sync_copy(
            cn_hbm.at[pl.ds(B * CTX + row0 * K, C * K)], neg_idx.at[b])
        pltpu.async_copy(o_emb.at[tgt_idx.at[b]], tgt_rows.at[b], sem_t.at[b])
        pltpu.async_copy(i_emb.at[ctx_idx.at[b]], ctx_rows.at[b], sem_c.at[b])
        pltpu.async_copy(o_emb.at[neg_idx.at[b]], neg_rows.at[b], sem_n.at[b])

    def drain(b):
        pltpu.make_async_copy(
            o_emb.at[tgt_idx.at[b]], tgt_rows.at[b], sem_t.at[b]).wait()
        pltpu.make_async_copy(
            i_emb.at[ctx_idx.at[b]], ctx_rows.at[b], sem_c.at[b]).wait()
        pltpu.make_async_copy(
            o_emb.at[neg_idx.at[b]], neg_rows.at[b], sem_n.at[b]).wait()

    lane = lax.iota(jnp.int32, 16)
    lane_sel = jnp.where(lane < S, lane, 0)
    sgn = jnp.where(lane == 0, 1.0, -1.0)
    valid = lane < S
    col_idx = [jnp.full((16,), j, jnp.int32) for j in range(16)]

    def compute(b, acc0):
        def row_body(c, acc):
            rc = c * CTX
            avg = []
            for q in range(D // 16):
                a = ctx_rows[b, rc, pl.ds(q * 16, 16)]
                for j in range(1, CTX):
                    a = a + ctx_rows[b, rc + j, pl.ds(q * 16, 16)]
                avg.append(a * (1.0 / CTX))
            p = tgt_rows[b, c, pl.ds(0, 16)] * avg[0]
            for q in range(1, D // 16):
                p = p + tgt_rows[b, c, pl.ds(q * 16, 16)] * avg[q]
            pbuf[0, :] = p
            rn = c * K
            for k in range(K):
                p = neg_rows[b, rn + k, pl.ds(0, 16)] * avg[0]
                for q in range(1, D // 16):
                    p = p + neg_rows[b, rn + k, pl.ds(q * 16, 16)] * avg[q]
                pbuf[k + 1, :] = p
            s = plsc.load_gather(pbuf, [lane_sel, col_idx[0]])
            for j in range(1, 16):
                s = s + plsc.load_gather(pbuf, [lane_sel, col_idx[j]])
            x = sgn * s
            y = jnp.exp(-jnp.abs(x))
            t = y / (y + 2.0)
            u = t * t
            poly = 1.0 + u * (1.0 / 3.0 + u * (1.0 / 5.0 + u * (1.0 / 7.0)))
            sp = jnp.maximum(-x, 0.0) + 2.0 * t * poly
            return acc + jnp.where(valid, sp, 0.0)

        return lax.fori_loop(0, C, row_body, acc0)

    fire(0, 0)
    acc = jnp.zeros((16,), jnp.float32)

    def body(i, acc):
        fire(2 * i + 1, 1)
        drain(0)
        acc = compute(0, acc)

        @pl.when(i < N_CHUNKS // 2 - 1)
        def _():
            fire(2 * i + 2, 0)

        drain(1)
        return compute(1, acc)

    acc = lax.fori_loop(0, N_CHUNKS // 2, body, acc)
    acc_buf[...] = acc
    pltpu.sync_copy(acc_buf, out_hbm.at[wid, :])


_sc_loss = functools.partial(
    pl.kernel,
    mesh=plsc.VectorSubcoreMesh(core_axis_name="c", subcore_axis_name="s"),
    compiler_params=pltpu.CompilerParams(
        needs_layout_passes=False, use_tc_tiling_on_sc=False
    ),
    out_type=jax.ShapeDtypeStruct((NW, 16), jnp.float32),
    scratch_types=[
        pltpu.VMEM((2, C), jnp.int32),
        pltpu.VMEM((2, C * CTX), jnp.int32),
        pltpu.VMEM((2, C * K), jnp.int32),
        pltpu.VMEM((2, C, D), jnp.float32),
        pltpu.VMEM((2, C * CTX, D), jnp.float32),
        pltpu.VMEM((2, C * K, D), jnp.float32),
        pltpu.VMEM((16, 16), jnp.float32),
        pltpu.VMEM((16,), jnp.float32),
        pltpu.SemaphoreType.DMA((2,)),
        pltpu.SemaphoreType.DMA((2,)),
        pltpu.SemaphoreType.DMA((2,)),
    ],
)(_sc_loss_kernel)


def _tc_loss_kernel(x_ref, o_ref):
    o_ref[0, 0] = jnp.sum(x_ref[...])


_tc_loss = pl.pallas_call(
    _tc_loss_kernel,
    out_shape=jax.ShapeDtypeStruct((1, 1), jnp.float32),
    out_specs=pl.BlockSpec(memory_space=pltpu.SMEM),
)


def kernel(i_emb, o_emb, target_wids, context_wids, neg_wids):
    tgt = target_wids.astype(jnp.int32)
    cn = jnp.concatenate(
        [context_wids.astype(jnp.int32), neg_wids.astype(jnp.int32)], axis=0
    ).reshape(-1)
    partials = _sc_loss(i_emb, o_emb, tgt, cn)
    loss = _tc_loss(partials)
    return loss[0, 0]


# speedup vs baseline: 2.2599x; 1.0415x over previous
import functools

import jax
import jax.numpy as jnp
from jax import lax
from jax.experimental import pallas as pl
from jax.experimental.pallas import tpu as pltpu
from jax.experimental.pallas import tpu_sc as plsc

V = 100000
D = 64
K = 10
CTX = 10
B = 16384
NC = 2
NS = 16
NW = NC * NS
BPW = B // NW
C = 32
N_CHUNKS = BPW // C
S = K + 1


def _tree_sum(vals):
    vals = list(vals)
    while len(vals) > 1:
        nxt = [a + b for a, b in zip(vals[0::2], vals[1::2])]
        if len(vals) % 2:
            nxt.append(vals[-1])
        vals = nxt
    return vals[0]


def _sc_loss_kernel(i_emb, o_emb, *refs):
    tgt_hbm = refs[0]
    ctx_hbm = refs[1:1 + CTX]
    neg_hbm = refs[1 + CTX:1 + CTX + K]
    (out_hbm, tgt_idx, ctx_idx, neg_idx, tgt_rows, ctx_rows, neg_rows,
     pbuf, acc_buf, sem_i, sem_t, sem_c, sem_n) = refs[1 + CTX + K:]

    wid = lax.axis_index("s") * NC + lax.axis_index("c")
    base = wid * BPW

    def fire(t, b):
        row0 = base + t * C
        sl = pl.ds(row0, C)
        pltpu.async_copy(tgt_hbm.at[sl], tgt_idx.at[b], sem_i.at[b])
        for j in range(CTX):
            pltpu.async_copy(
                ctx_hbm[j].at[sl], ctx_idx.at[b, pl.ds(j * C, C)], sem_i.at[b])
        for j in range(K):
            pltpu.async_copy(
                neg_hbm[j].at[sl], neg_idx.at[b, pl.ds(j * C, C)], sem_i.at[b])
        pltpu.make_async_copy(tgt_hbm.at[sl], tgt_idx.at[b], sem_i.at[b]).wait()
        for j in range(CTX):
            pltpu.make_async_copy(
                ctx_hbm[j].at[sl], ctx_idx.at[b, pl.ds(j * C, C)],
                sem_i.at[b]).wait()
        for j in range(K):
            pltpu.make_async_copy(
                neg_hbm[j].at[sl], neg_idx.at[b, pl.ds(j * C, C)],
                sem_i.at[b]).wait()
        pltpu.async_copy(o_emb.at[tgt_idx.at[b]], tgt_rows.at[b], sem_t.at[b])
        pltpu.async_copy(i_emb.at[ctx_idx.at[b]], ctx_rows.at[b], sem_c.at[b])
        pltpu.async_copy(o_emb.at[neg_idx.at[b]], neg_rows.at[b], sem_n.at[b])

    def drain(b):
        pltpu.make_async_copy(
            o_emb.at[tgt_idx.at[b]], tgt_rows.at[b], sem_t.at[b]).wait()
        pltpu.make_async_copy(
            i_emb.at[ctx_idx.at[b]], ctx_rows.at[b], sem_c.at[b]).wait()
        pltpu.make_async_copy(
            o_emb.at[neg_idx.at[b]], neg_rows.at[b], sem_n.at[b]).wait()

    lane = lax.iota(jnp.int32, 16)
    lane_sel = jnp.where(lane < S, lane, 0)
    sgn = jnp.where(lane == 0, 1.0, -1.0)
    valid = lane < S
    col_idx = [jnp.full((16,), j, jnp.int32) for j in range(16)]

    def compute(b, acc0):
        def row_body(c, acc):
            avg = []
            for q in range(D // 16):
                a = _tree_sum(
                    [ctx_rows[b, j * C + c, pl.ds(q * 16, 16)]
                     for j in range(CTX)])
                avg.append(a * (1.0 / CTX))
            pbuf[0, :] = _tree_sum(
                [tgt_rows[b, c, pl.ds(q * 16, 16)] * avg[q]
                 for q in range(D // 16)])
            for k in range(K):
                pbuf[k + 1, :] = _tree_sum(
                    [neg_rows[b, k * C + c, pl.ds(q * 16, 16)] * avg[q]
                     for q in range(D // 16)])
            s = _tree_sum(
                [plsc.load_gather(pbuf, [lane_sel, col_idx[j]])
                 for j in range(16)])
            x = sgn * s
            y = jnp.exp(-jnp.abs(x))
            t = y / (y + 2.0)
            u = t * t
            poly = 1.0 + u * (1.0 / 3.0 + u * (1.0 / 5.0 + u * (1.0 / 7.0)))
            sp = jnp.maximum(-x, 0.0) + 2.0 * t * poly
            return acc + jnp.where(valid, sp, 0.0)

        return lax.fori_loop(0, C, row_body, acc0)

    fire(0, 0)
    acc = jnp.zeros((16,), jnp.float32)

    def body(i, acc):
        fire(2 * i + 1, 1)
        drain(0)
        acc = compute(0, acc)

        @pl.when(i < N_CHUNKS // 2 - 1)
        def _():
            fire(2 * i + 2, 0)

        drain(1)
        return compute(1, acc)

    acc = lax.fori_loop(0, N_CHUNKS // 2, body, acc)
    acc_buf[...] = acc
    pltpu.sync_copy(acc_buf, out_hbm.at[wid, :])


_sc_loss = functools.partial(
    pl.kernel,
    mesh=plsc.VectorSubcoreMesh(core_axis_name="c", subcore_axis_name="s"),
    compiler_params=pltpu.CompilerParams(
        needs_layout_passes=False, use_tc_tiling_on_sc=False
    ),
    out_type=jax.ShapeDtypeStruct((NW, 16), jnp.float32),
    scratch_types=[
        pltpu.VMEM((2, C), jnp.int32),
        pltpu.VMEM((2, C * CTX), jnp.int32),
        pltpu.VMEM((2, C * K), jnp.int32),
        pltpu.VMEM((2, C, D), jnp.float32),
        pltpu.VMEM((2, C * CTX, D), jnp.float32),
        pltpu.VMEM((2, C * K, D), jnp.float32),
        pltpu.VMEM((16, 16), jnp.float32),
        pltpu.VMEM((16,), jnp.float32),
        pltpu.SemaphoreType.DMA((2,)),
        pltpu.SemaphoreType.DMA((2,)),
        pltpu.SemaphoreType.DMA((2,)),
        pltpu.SemaphoreType.DMA((2,)),
    ],
)(_sc_loss_kernel)


def _tc_loss_kernel(x_ref, o_ref):
    o_ref[0, 0] = jnp.sum(x_ref[...])


_tc_loss = pl.pallas_call(
    _tc_loss_kernel,
    out_shape=jax.ShapeDtypeStruct((1, 1), jnp.float32),
    out_specs=pl.BlockSpec(memory_space=pltpu.SMEM),
)


def kernel(i_emb, o_emb, target_wids, context_wids, neg_wids):
    tgt = target_wids.astype(jnp.int32)
    ctx_cols = [context_wids[:, j].astype(jnp.int32) for j in range(CTX)]
    neg_cols = [neg_wids[:, j].astype(jnp.int32) for j in range(K)]
    partials = _sc_loss(i_emb, o_emb, tgt, *ctx_cols, *neg_cols)
    loss = _tc_loss(partials)
    return loss[0, 0]
